# baseline (speedup 1.0000x reference)
import functools

import jax
import jax.numpy as jnp
from jax import lax
from jax.experimental import pallas as pl
from jax.experimental.pallas import tpu as pltpu
from jax.experimental.pallas import tpu_sc as plsc

NC, NS, L = 2, 16, 16
NW = NC * NS
R, C = 128, 32768
K = 16
HC = C // 2
BR = 32

_MESH = plsc.VectorSubcoreMesh(
    core_axis_name="c", subcore_axis_name="s", num_cores=NC, num_subcores=NS
)

_KEY_BASE = 0x3E19999A


def _tc_score_body(x_ref, out_ref):
    i = pl.program_id(0)
    x = x_ref[...]
    s = jnp.sum(x, axis=1)
    q = jnp.sum(x * x, axis=1)
    var = (q - s * s * (1.0 / C)) * (1.0 / (C - 1))
    std = jnp.sqrt(jnp.maximum(var, 0.0))
    realism = jnp.where(
        std < 0.01,
        std * 10.0,
        jnp.where(std > 0.5, 0.5 / std, 1.0 - jnp.abs(std - 0.1)),
    )
    out_ref[pl.ds(i, 1)] = (realism + 0.15).reshape(1, 1, BR)


_score_tc = pl.pallas_call(
    _tc_score_body,
    grid=(R // BR,),
    in_specs=[pl.BlockSpec((BR, C), lambda i: (i, 0))],
    out_specs=pl.BlockSpec((R // BR, 1, BR), lambda i: (0, 0, 0)),
    out_shape=jax.ShapeDtypeStruct((R // BR, 1, BR), jnp.float32),
    compiler_params=pltpu.CompilerParams(dimension_semantics=("arbitrary",)),
)


def _lane_iota():
    return lax.iota(jnp.int32, L)


def _gather_scores(sraw, rows):
    return plsc.load_gather(
        sraw,
        [
            lax.shift_right_arithmetic(rows, jnp.full((L,), 5, jnp.int32)),
            jnp.full((L,), 0, jnp.int32),
            lax.bitwise_and(rows, jnp.full((L,), BR - 1, jnp.int32)),
        ],
    )


@functools.partial(
    pl.kernel,
    out_type=(
        jax.ShapeDtypeStruct((K,), jnp.int32),
        jax.ShapeDtypeStruct((K,), jnp.float32),
    ),
    mesh=_MESH,
    scratch_types=[
        pltpu.VMEM((R // BR, 1, BR), jnp.float32),
        pltpu.VMEM((K,), jnp.int32),
        pltpu.VMEM((K,), jnp.float32),
    ],
    compiler_params=pltpu.CompilerParams(needs_layout_passes=False),
)
def _topk_stage(scores_hbm, oidx_hbm, oscores_hbm, sraw, tidx, tsc):
    wid = lax.axis_index("s") * NC + lax.axis_index("c")
    lane = _lane_iota()

    pltpu.sync_copy(scores_hbm, sraw)

    pairs = []
    for v in range(8):
        jv = lane + (16 * v)
        sv = _gather_scores(sraw, jv)
        bits = lax.bitcast_convert_type(sv, jnp.uint32)
        diff = bits - jnp.full((L,), _KEY_BASE, jnp.uint32)
        key = lax.bitwise_or(
            lax.shift_left(diff, jnp.full((L,), 7, jnp.uint32)),
            lax.bitcast_convert_type(jnp.full((L,), 127, jnp.int32) - jv,
                                     jnp.uint32),
        )
        pairs.append(plsc.sort_key_val(key, jv, descending=True))

    def merge(a, b):
        ka, va = a
        kb, vb = b
        kr = lax.rev(kb, (0,))
        vr = lax.rev(vb, (0,))
        m = ka >= kr
        kk = jnp.where(m, ka, kr)
        vv = jnp.where(m, va, vr)
        return plsc.sort_key_val(kk, vv, descending=True)

    while len(pairs) > 1:
        pairs = [merge(pairs[i], pairs[i + 1]) for i in range(0, len(pairs), 2)]
    _, top_rows = pairs[0]

    @pl.when(wid == 0)
    def _():
        tidx[...] = top_rows
        tsc[...] = _gather_scores(sraw, top_rows)
        pltpu.sync_copy(tidx, oidx_hbm)
        pltpu.sync_copy(tsc, oscores_hbm)


def _tc_gather_body(idx_ref, x_hbm, o_hbm, sem):
    copies = [
        pltpu.make_async_copy(
            x_hbm.at[pl.ds(idx_ref[i], 1)], o_hbm.at[pl.ds(i, 1)], sem
        )
        for i in range(K)
    ]
    for cp in copies:
        cp.start()
    for cp in copies:
        cp.wait()


_gather_tc = pl.pallas_call(
    _tc_gather_body,
    grid_spec=pltpu.PrefetchScalarGridSpec(
        num_scalar_prefetch=1,
        grid=(1,),
        in_specs=[pl.BlockSpec(memory_space=pl.ANY)],
        out_specs=pl.BlockSpec(memory_space=pl.ANY),
        scratch_shapes=[pltpu.SemaphoreType.DMA],
    ),
    out_shape=jax.ShapeDtypeStruct((K, C), jnp.float32),
)


def kernel(chunks, regime_probs, keep_top_k):
    del regime_probs, keep_top_k
    scores = _score_tc(chunks)
    top_idx, top_scores = _topk_stage(scores)
    pruned = _gather_tc(top_idx, chunks)
    return (pruned, top_scores)


# speedup vs baseline: 3.2727x; 3.2727x over previous
import functools

import jax
import jax.numpy as jnp
from jax import lax
from jax.experimental import pallas as pl
from jax.experimental.pallas import tpu as pltpu
from jax.experimental.pallas import tpu_sc as plsc

NC, NS, L = 2, 16, 16
NW = NC * NS
R, C = 128, 32768
K = 16
HC = C // 2
BR = 32

_MESH = plsc.VectorSubcoreMesh(
    core_axis_name="c", subcore_axis_name="s", num_cores=1, num_subcores=NS
)

_KEY_BASE = 0x3E19999A


def _tc_score_body(x_ref, out_ref):
    i = pl.program_id(0)
    x = x_ref[...]
    s = jnp.sum(x, axis=1)
    q = jnp.sum(x * x, axis=1)
    var = (q - s * s * (1.0 / C)) * (1.0 / (C - 1))
    std = jnp.sqrt(jnp.maximum(var, 0.0))
    realism = jnp.where(
        std < 0.01,
        std * 10.0,
        jnp.where(std > 0.5, 0.5 / std, 1.0 - jnp.abs(std - 0.1)),
    )
    out_ref[pl.ds(i, 1)] = (realism + 0.15).reshape(1, 1, BR)


_score_tc = pl.pallas_call(
    _tc_score_body,
    grid=(R // BR,),
    in_specs=[pl.BlockSpec((BR, C), lambda i: (i, 0))],
    out_specs=pl.BlockSpec((R // BR, 1, BR), lambda i: (0, 0, 0)),
    out_shape=jax.ShapeDtypeStruct((R // BR, 1, BR), jnp.float32),
    compiler_params=pltpu.CompilerParams(dimension_semantics=("arbitrary",)),
)


def _lane_iota():
    return lax.iota(jnp.int32, L)


def _gather_scores(sraw, rows):
    return plsc.load_gather(
        sraw,
        [
            lax.shift_right_arithmetic(rows, jnp.full((L,), 5, jnp.int32)),
            jnp.full((L,), 0, jnp.int32),
            lax.bitwise_and(rows, jnp.full((L,), BR - 1, jnp.int32)),
        ],
    )


@functools.partial(
    pl.kernel,
    out_type=(
        jax.ShapeDtypeStruct((K,), jnp.int32),
        jax.ShapeDtypeStruct((K,), jnp.float32),
    ),
    mesh=_MESH,
    scratch_types=[
        pltpu.VMEM((R // BR, 1, BR), jnp.float32),
        pltpu.VMEM((K,), jnp.int32),
        pltpu.VMEM((K,), jnp.float32),
    ],
    compiler_params=pltpu.CompilerParams(needs_layout_passes=False),
)
def _topk_stage(scores_hbm, oidx_hbm, oscores_hbm, sraw, tidx, tsc):
    wid = lax.axis_index("s")
    lane = _lane_iota()

    pltpu.sync_copy(scores_hbm, sraw)

    pairs = []
    for v in range(8):
        jv = lane + (16 * v)
        sv = _gather_scores(sraw, jv)
        bits = lax.bitcast_convert_type(sv, jnp.uint32)
        diff = bits - jnp.full((L,), _KEY_BASE, jnp.uint32)
        key = lax.bitwise_or(
            lax.shift_left(diff, jnp.full((L,), 7, jnp.uint32)),
            lax.bitcast_convert_type(jnp.full((L,), 127, jnp.int32) - jv,
                                     jnp.uint32),
        )
        pairs.append(plsc.sort_key_val(key, jv, descending=True))

    def merge(a, b):
        ka, va = a
        kb, vb = b
        kr = lax.rev(kb, (0,))
        vr = lax.rev(vb, (0,))
        m = ka >= kr
        kk = jnp.where(m, ka, kr)
        vv = jnp.where(m, va, vr)
        return plsc.sort_key_val(kk, vv, descending=True)

    while len(pairs) > 1:
        pairs = [merge(pairs[i], pairs[i + 1]) for i in range(0, len(pairs), 2)]
    _, top_rows = pairs[0]

    @pl.when(wid == 0)
    def _():
        tidx[...] = top_rows
        tsc[...] = _gather_scores(sraw, top_rows)
        pltpu.sync_copy(tidx, oidx_hbm)
        pltpu.sync_copy(tsc, oscores_hbm)


def _tc_gather_body(idx_ref, x_hbm, o_hbm, buf, semI, semO):
    ins = [
        pltpu.make_async_copy(
            x_hbm.at[pl.ds(idx_ref[i], 1)], buf.at[pl.ds(i, 1)], semI.at[i]
        )
        for i in range(K)
    ]
    outs = [
        pltpu.make_async_copy(
            buf.at[pl.ds(i, 1)], o_hbm.at[pl.ds(i, 1)], semO
        )
        for i in range(K)
    ]
    for cp in ins:
        cp.start()
    for i in range(K):
        ins[i].wait()
        outs[i].start()
    for cp in outs:
        cp.wait()


_gather_tc = pl.pallas_call(
    _tc_gather_body,
    grid_spec=pltpu.PrefetchScalarGridSpec(
        num_scalar_prefetch=1,
        grid=(1,),
        in_specs=[pl.BlockSpec(memory_space=pl.ANY)],
        out_specs=pl.BlockSpec(memory_space=pl.ANY),
        scratch_shapes=[
            pltpu.VMEM((K, C), jnp.float32),
            pltpu.SemaphoreType.DMA((K,)),
            pltpu.SemaphoreType.DMA,
        ],
    ),
    out_shape=jax.ShapeDtypeStruct((K, C), jnp.float32),
)


def kernel(chunks, regime_probs, keep_top_k):
    del regime_probs, keep_top_k
    scores = _score_tc(chunks)
    top_idx, top_scores = _topk_stage(scores)
    pruned = _gather_tc(top_idx, chunks)
    return (pruned, top_scores)
